# Initial kernel scaffold; baseline (speedup 1.0000x reference)
import jax
import jax.numpy as jnp
from jax.experimental import pallas as pl


def kernel(inp, input_grid, output_grid, neighbors_index, W1, b1, W2, b2, K1, kb1, K2, kb2):
    raise NotImplementedError("write your pallas kernel here")


# speedup vs baseline: 5.4906x; 5.4906x over previous
import functools

import jax
import jax.numpy as jnp
from jax import lax
from jax.experimental import pallas as pl
from jax.experimental.pallas import tpu as pltpu
from jax.experimental.pallas import tpu_sc as plsc

N_IN = 10000
N_OUT = 10000
K = 32
E = N_OUT * K
IN_DIM = 128
OUT_DIM = 128
PROJ_HID = 256
KER_HID = 32
NDIM = 3
YPAD = 128

_PROJ_BLK = 1000


def _proj_body(x_ref, w1_ref, b1_ref, w2_ref, b2_ref, o_ref):
    h = jnp.dot(x_ref[...], w1_ref[...], preferred_element_type=jnp.float32)
    h = jax.nn.gelu(h + b1_ref[...])
    o_ref[...] = (
        jnp.dot(h, w2_ref[...], preferred_element_type=jnp.float32) + b2_ref[...]
    )


def _proj(x, w1, b1, w2, b2):
    grid = (N_IN // _PROJ_BLK,)
    return pl.pallas_call(
        _proj_body,
        grid=grid,
        in_specs=[
            pl.BlockSpec((_PROJ_BLK, IN_DIM), lambda i: (i, 0)),
            pl.BlockSpec((IN_DIM, PROJ_HID), lambda i: (0, 0)),
            pl.BlockSpec((PROJ_HID,), lambda i: (0,)),
            pl.BlockSpec((PROJ_HID, OUT_DIM), lambda i: (0, 0)),
            pl.BlockSpec((OUT_DIM,), lambda i: (0,)),
        ],
        out_specs=pl.BlockSpec((_PROJ_BLK, OUT_DIM), lambda i: (i, 0)),
        out_shape=jax.ShapeDtypeStruct((N_IN, OUT_DIM), jnp.float32),
    )(x, w1, b1, w2, b2)


_NC = 2
_NS = 16
_NW = _NC * _NS
_PER_W = E // _NW
_CHUNK = 80
_NCHUNK = _PER_W // _CHUNK


def _gather_body(idx_hbm, grid_hbm, f_hbm, y_out, g_out,
                 idx_v, y_v, g_v, sy, sg):
    wid = lax.axis_index("s") * _NC + lax.axis_index("c")
    base = wid * _PER_W

    def body(t, carry):
        off = base + t * _CHUNK
        pltpu.sync_copy(idx_hbm.at[pl.ds(off, _CHUNK)], idx_v)
        cy = pltpu.async_copy(grid_hbm.at[idx_v], y_v, sy)
        cg = pltpu.async_copy(f_hbm.at[idx_v], g_v, sg)
        cy.wait()
        cg.wait()
        pltpu.sync_copy(y_v, y_out.at[pl.ds(off, _CHUNK)])
        pltpu.sync_copy(g_v, g_out.at[pl.ds(off, _CHUNK)])
        return carry

    lax.fori_loop(0, _NCHUNK, body, 0)


def _gather(idx, gridp, f):
    mesh = plsc.VectorSubcoreMesh(core_axis_name="c", subcore_axis_name="s")
    return pl.kernel(
        _gather_body,
        out_type=(
            jax.ShapeDtypeStruct((E, YPAD), jnp.float32),
            jax.ShapeDtypeStruct((E, IN_DIM), jnp.float32),
        ),
        mesh=mesh,
        scratch_types=[
            pltpu.VMEM((_CHUNK,), jnp.int32),
            pltpu.VMEM((_CHUNK, YPAD), jnp.float32),
            pltpu.VMEM((_CHUNK, IN_DIM), jnp.float32),
            pltpu.SemaphoreType.DMA,
            pltpu.SemaphoreType.DMA,
        ],
    )(idx, gridp, f)


_BPTS = 400
_BE = _BPTS * K


def _edge_body(y_ref, g_ref, og_ref, k1a_ref, k1b_ref, kb1_ref,
               k2_ref, kb2_ref, o_ref):
    a1 = jnp.dot(y_ref[...], k1a_ref[...], preferred_element_type=jnp.float32)
    q = (
        jnp.dot(og_ref[...], k1b_ref[...], preferred_element_type=jnp.float32)
        + kb1_ref[...]
    )
    h1 = jax.nn.gelu(a1.reshape(_BPTS, K, KER_HID) + q[:, None, :])
    kern = (
        jnp.dot(h1.reshape(_BE, KER_HID), k2_ref[...],
                preferred_element_type=jnp.float32)
        + kb2_ref[...]
    )
    vals = kern * g_ref[...]
    o_ref[...] = vals.reshape(_BPTS, K, OUT_DIM).sum(axis=1) * (1.0 / K)


def _edge(y, g, ogp, k1a, k1b, kb1, k2, kb2):
    grid = (N_OUT // _BPTS,)
    return pl.pallas_call(
        _edge_body,
        grid=grid,
        in_specs=[
            pl.BlockSpec((_BE, YPAD), lambda i: (i, 0)),
            pl.BlockSpec((_BE, OUT_DIM), lambda i: (i, 0)),
            pl.BlockSpec((_BPTS, 4), lambda i: (i, 0)),
            pl.BlockSpec((YPAD, KER_HID), lambda i: (0, 0)),
            pl.BlockSpec((4, KER_HID), lambda i: (0, 0)),
            pl.BlockSpec((KER_HID,), lambda i: (0,)),
            pl.BlockSpec((KER_HID, OUT_DIM), lambda i: (0, 0)),
            pl.BlockSpec((OUT_DIM,), lambda i: (0,)),
        ],
        out_specs=pl.BlockSpec((_BPTS, OUT_DIM), lambda i: (i, 0)),
        out_shape=jax.ShapeDtypeStruct((N_OUT, OUT_DIM), jnp.float32),
    )(y, g, ogp, k1a, k1b, kb1, k2, kb2)


def kernel(inp, input_grid, output_grid, neighbors_index,
           W1, b1, W2, b2, K1, kb1, K2, kb2):
    x = inp[0]
    f = _proj(x, W1, b1, W2, b2)

    gridp = jnp.zeros((N_IN, YPAD), jnp.float32).at[:, :NDIM].set(input_grid)
    y, g = _gather(neighbors_index, gridp, f)

    ogp = jnp.pad(output_grid, ((0, 0), (0, 1)))
    k1a = jnp.zeros((YPAD, KER_HID), jnp.float32).at[:NDIM, :].set(K1[:NDIM])
    k1b = jnp.zeros((4, KER_HID), jnp.float32).at[:NDIM, :].set(K1[NDIM:])

    out = _edge(y, g, ogp, k1a, k1b, kb1, K2, kb2)
    return out[None]


# speedup vs baseline: 6.0969x; 1.1104x over previous
import functools

import jax
import jax.numpy as jnp
from jax import lax
from jax.experimental import pallas as pl
from jax.experimental.pallas import tpu as pltpu
from jax.experimental.pallas import tpu_sc as plsc

N_IN = 10000
N_OUT = 10000
K = 32
E = N_OUT * K
IN_DIM = 128
OUT_DIM = 128
PROJ_HID = 256
KER_HID = 32
NDIM = 3
TW = 128
FH = OUT_DIM // 2
PH = KER_HID // 2


def _pack_bf16(hi, lo):
    uh = lax.bitcast_convert_type(hi.astype(jnp.bfloat16).astype(jnp.float32),
                                  jnp.uint32)
    ul = lax.bitcast_convert_type(lo.astype(jnp.bfloat16).astype(jnp.float32),
                                  jnp.uint32)
    return lax.bitcast_convert_type(uh | (ul >> 16), jnp.float32)


_PROJ_BLK = 1000


def _proj_body(x_ref, g_ref, w1_ref, b1_ref, w2_ref, b2_ref, k1a_ref, o_ref):
    h = jnp.dot(x_ref[...], w1_ref[...], preferred_element_type=jnp.float32)
    h = jax.nn.gelu(h + b1_ref[...])
    f = jnp.dot(h, w2_ref[...], preferred_element_type=jnp.float32) + b2_ref[...]
    p = jnp.dot(g_ref[...], k1a_ref[...], preferred_element_type=jnp.float32)
    pf = _pack_bf16(f[:, :FH], f[:, FH:])
    pp = _pack_bf16(p[:, :PH], p[:, PH:])
    zz = jnp.zeros((_PROJ_BLK, TW - FH - PH), jnp.float32)
    o_ref[...] = jnp.concatenate([pf, pp, zz], axis=1)


def _proj_pack(x, gridp, w1, b1, w2, b2, k1a):
    grid = (N_IN // _PROJ_BLK,)
    return pl.pallas_call(
        _proj_body,
        grid=grid,
        in_specs=[
            pl.BlockSpec((_PROJ_BLK, IN_DIM), lambda i: (i, 0)),
            pl.BlockSpec((_PROJ_BLK, 4), lambda i: (i, 0)),
            pl.BlockSpec((IN_DIM, PROJ_HID), lambda i: (0, 0)),
            pl.BlockSpec((PROJ_HID,), lambda i: (0,)),
            pl.BlockSpec((PROJ_HID, OUT_DIM), lambda i: (0, 0)),
            pl.BlockSpec((OUT_DIM,), lambda i: (0,)),
            pl.BlockSpec((4, KER_HID), lambda i: (0, 0)),
        ],
        out_specs=pl.BlockSpec((_PROJ_BLK, TW), lambda i: (i, 0)),
        out_shape=jax.ShapeDtypeStruct((N_IN, TW), jnp.float32),
    )(x, gridp, w1, b1, w2, b2, k1a)


_NC = 2
_NS = 16
_NW = _NC * _NS
_PER_W = E // _NW
_CHUNK = 80
_NCHUNK = _PER_W // _CHUNK


def _gather_body(idx_hbm, t_hbm, t_out, idx_v, t_v, st):
    wid = lax.axis_index("s") * _NC + lax.axis_index("c")
    base = wid * _PER_W

    def body(t, carry):
        off = base + t * _CHUNK
        pltpu.sync_copy(idx_hbm.at[pl.ds(off, _CHUNK)], idx_v)
        pltpu.async_copy(t_hbm.at[idx_v], t_v, st).wait()
        pltpu.sync_copy(t_v, t_out.at[pl.ds(off, _CHUNK)])
        return carry

    lax.fori_loop(0, _NCHUNK, body, 0)


def _gather(idx, table):
    mesh = plsc.VectorSubcoreMesh(core_axis_name="c", subcore_axis_name="s")
    return pl.kernel(
        _gather_body,
        out_type=jax.ShapeDtypeStruct((E, TW), jnp.float32),
        mesh=mesh,
        scratch_types=[
            pltpu.VMEM((_CHUNK,), jnp.int32),
            pltpu.VMEM((_CHUNK, TW), jnp.float32),
            pltpu.SemaphoreType.DMA,
        ],
    )(idx, table)


_BPTS = 400
_BE = _BPTS * K


def _edge_body(t_ref, og_ref, k1b_ref, kb1_ref, k2_ref, kb2_ref, o_ref):
    u = lax.bitcast_convert_type(t_ref[...], jnp.uint32)
    hi = lax.bitcast_convert_type(u & jnp.uint32(0xFFFF0000), jnp.float32)
    lo = lax.bitcast_convert_type(u << 16, jnp.float32)
    g = jnp.concatenate([hi[:, :FH], lo[:, :FH]], axis=1)
    a1 = jnp.concatenate([hi[:, FH:FH + PH], lo[:, FH:FH + PH]], axis=1)
    q = (
        jnp.dot(og_ref[...], k1b_ref[...], preferred_element_type=jnp.float32)
        + kb1_ref[...]
    )
    h1 = jax.nn.gelu(a1.reshape(_BPTS, K, KER_HID) + q[:, None, :])
    kern = (
        jnp.dot(h1.reshape(_BE, KER_HID), k2_ref[...],
                preferred_element_type=jnp.float32)
        + kb2_ref[...]
    )
    vals = kern * g
    o_ref[...] = vals.reshape(_BPTS, K, OUT_DIM).sum(axis=1) * (1.0 / K)


def _edge(tg, ogp, k1b, kb1, k2, kb2):
    grid = (N_OUT // _BPTS,)
    return pl.pallas_call(
        _edge_body,
        grid=grid,
        in_specs=[
            pl.BlockSpec((_BE, TW), lambda i: (i, 0)),
            pl.BlockSpec((_BPTS, 4), lambda i: (i, 0)),
            pl.BlockSpec((4, KER_HID), lambda i: (0, 0)),
            pl.BlockSpec((KER_HID,), lambda i: (0,)),
            pl.BlockSpec((KER_HID, OUT_DIM), lambda i: (0, 0)),
            pl.BlockSpec((OUT_DIM,), lambda i: (0,)),
        ],
        out_specs=pl.BlockSpec((_BPTS, OUT_DIM), lambda i: (i, 0)),
        out_shape=jax.ShapeDtypeStruct((N_OUT, OUT_DIM), jnp.float32),
    )(tg, ogp, k1b, kb1, k2, kb2)


def kernel(inp, input_grid, output_grid, neighbors_index,
           W1, b1, W2, b2, K1, kb1, K2, kb2):
    x = inp[0]
    gridp = jnp.pad(input_grid, ((0, 0), (0, 1)))
    k1a = jnp.zeros((4, KER_HID), jnp.float32).at[:NDIM, :].set(K1[:NDIM])
    k1b = jnp.zeros((4, KER_HID), jnp.float32).at[:NDIM, :].set(K1[NDIM:])

    table = _proj_pack(x, gridp, W1, b1, W2, b2, k1a)
    tg = _gather(neighbors_index, table)

    ogp = jnp.pad(output_grid, ((0, 0), (0, 1)))
    out = _edge(tg, ogp, k1b, kb1, K2, kb2)
    return out[None]


# speedup vs baseline: 8.6662x; 1.4214x over previous
import functools

import jax
import jax.numpy as jnp
from jax import lax
from jax.experimental import pallas as pl
from jax.experimental.pallas import tpu as pltpu
from jax.experimental.pallas import tpu_sc as plsc

N_IN = 10000
N_OUT = 10000
K = 32
E = N_OUT * K
IN_DIM = 128
OUT_DIM = 128
PROJ_HID = 256
KER_HID = 32
NDIM = 3
TW = 128
FH = OUT_DIM // 2
PH = KER_HID // 2


def _pack_bf16(hi, lo):
    uh = lax.bitcast_convert_type(hi.astype(jnp.bfloat16).astype(jnp.float32),
                                  jnp.uint32)
    ul = lax.bitcast_convert_type(lo.astype(jnp.bfloat16).astype(jnp.float32),
                                  jnp.uint32)
    return lax.bitcast_convert_type(uh | (ul >> 16), jnp.float32)


_PROJ_BLK = 1000


def _proj_body(x_ref, g_ref, w1_ref, b1_ref, w2_ref, b2_ref, k1a_ref, o_ref):
    h = jnp.dot(x_ref[...], w1_ref[...], preferred_element_type=jnp.float32)
    h = jax.nn.gelu(h + b1_ref[...])
    f = jnp.dot(h, w2_ref[...], preferred_element_type=jnp.float32) + b2_ref[...]
    p = jnp.dot(g_ref[...], k1a_ref[...], preferred_element_type=jnp.float32)
    pf = _pack_bf16(f[:, :FH], f[:, FH:])
    pp = _pack_bf16(p[:, :PH], p[:, PH:])
    zz = jnp.zeros((_PROJ_BLK, TW - FH - PH), jnp.float32)
    o_ref[...] = jnp.concatenate([pf, pp, zz], axis=1)


def _proj_pack(x, gridp, w1, b1, w2, b2, k1a):
    grid = (N_IN // _PROJ_BLK,)
    return pl.pallas_call(
        _proj_body,
        grid=grid,
        in_specs=[
            pl.BlockSpec((_PROJ_BLK, IN_DIM), lambda i: (i, 0)),
            pl.BlockSpec((_PROJ_BLK, 4), lambda i: (i, 0)),
            pl.BlockSpec((IN_DIM, PROJ_HID), lambda i: (0, 0)),
            pl.BlockSpec((PROJ_HID,), lambda i: (0,)),
            pl.BlockSpec((PROJ_HID, OUT_DIM), lambda i: (0, 0)),
            pl.BlockSpec((OUT_DIM,), lambda i: (0,)),
            pl.BlockSpec((4, KER_HID), lambda i: (0, 0)),
        ],
        out_specs=pl.BlockSpec((_PROJ_BLK, TW), lambda i: (i, 0)),
        out_shape=jax.ShapeDtypeStruct((N_IN, TW), jnp.float32),
    )(x, gridp, w1, b1, w2, b2, k1a)


_NC = 2
_NS = 16
_NW = _NC * _NS
_PER_W = E // _NW
_CHUNK = 80
_NCHUNK = _PER_W // _CHUNK


def _gather_body(idx_hbm, t_hbm, t_out, idx_v, t_v, sg, so):
    wid = lax.axis_index("s") * _NC + lax.axis_index("c")
    base = wid * _PER_W

    pltpu.sync_copy(idx_hbm.at[pl.ds(base, _PER_W)], idx_v)

    def start_gather(t, buf):
        pltpu.async_copy(
            t_hbm.at[idx_v.at[pl.ds(t * _CHUNK, _CHUNK)]], t_v.at[buf], sg)

    def drain_gather(buf):
        pltpu.make_async_copy(
            t_hbm.at[idx_v.at[pl.ds(0, _CHUNK)]], t_v.at[buf], sg).wait()

    def drain_store(buf):
        pltpu.make_async_copy(
            t_v.at[buf], t_out.at[pl.ds(base, _CHUNK)], so).wait()

    start_gather(0, 0)

    def body(t, carry):
        cur = lax.rem(t, 2)
        nxt = lax.rem(t + 1, 2)

        @pl.when(t + 1 < _NCHUNK)
        def _():
            @pl.when(t >= 1)
            def _():
                drain_store(nxt)
            start_gather(t + 1, nxt)

        drain_gather(cur)
        pltpu.async_copy(
            t_v.at[cur], t_out.at[pl.ds(base + t * _CHUNK, _CHUNK)], so)
        return carry

    lax.fori_loop(0, _NCHUNK, body, 0)
    drain_store(0)
    drain_store(1)


def _gather(idx, table):
    mesh = plsc.VectorSubcoreMesh(core_axis_name="c", subcore_axis_name="s")
    return pl.kernel(
        _gather_body,
        out_type=jax.ShapeDtypeStruct((E, TW), jnp.float32),
        mesh=mesh,
        scratch_types=[
            pltpu.VMEM((_PER_W,), jnp.int32),
            pltpu.VMEM((2, _CHUNK, TW), jnp.float32),
            pltpu.SemaphoreType.DMA,
            pltpu.SemaphoreType.DMA,
        ],
    )(idx, table)


_BPTS = 400
_BE = _BPTS * K


def _edge_body(t_ref, og_ref, k1b_ref, kb1_ref, k2_ref, kb2_ref, o_ref):
    u = lax.bitcast_convert_type(t_ref[...], jnp.uint32)
    hi = lax.bitcast_convert_type(u & jnp.uint32(0xFFFF0000), jnp.float32)
    lo = lax.bitcast_convert_type(u << 16, jnp.float32)
    g = jnp.concatenate([hi[:, :FH], lo[:, :FH]], axis=1)
    a1 = jnp.concatenate([hi[:, FH:FH + PH], lo[:, FH:FH + PH]], axis=1)
    q = (
        jnp.dot(og_ref[...], k1b_ref[...], preferred_element_type=jnp.float32)
        + kb1_ref[...]
    )
    h1 = jax.nn.gelu(a1.reshape(_BPTS, K, KER_HID) + q[:, None, :])
    kern = (
        jnp.dot(h1.reshape(_BE, KER_HID), k2_ref[...],
                preferred_element_type=jnp.float32)
        + kb2_ref[...]
    )
    vals = kern * g
    o_ref[...] = vals.reshape(_BPTS, K, OUT_DIM).sum(axis=1) * (1.0 / K)


def _edge(tg, ogp, k1b, kb1, k2, kb2):
    grid = (N_OUT // _BPTS,)
    return pl.pallas_call(
        _edge_body,
        grid=grid,
        in_specs=[
            pl.BlockSpec((_BE, TW), lambda i: (i, 0)),
            pl.BlockSpec((_BPTS, 4), lambda i: (i, 0)),
            pl.BlockSpec((4, KER_HID), lambda i: (0, 0)),
            pl.BlockSpec((KER_HID,), lambda i: (0,)),
            pl.BlockSpec((KER_HID, OUT_DIM), lambda i: (0, 0)),
            pl.BlockSpec((OUT_DIM,), lambda i: (0,)),
        ],
        out_specs=pl.BlockSpec((_BPTS, OUT_DIM), lambda i: (i, 0)),
        out_shape=jax.ShapeDtypeStruct((N_OUT, OUT_DIM), jnp.float32),
    )(tg, ogp, k1b, kb1, k2, kb2)


def kernel(inp, input_grid, output_grid, neighbors_index,
           W1, b1, W2, b2, K1, kb1, K2, kb2):
    x = inp[0]
    gridp = jnp.pad(input_grid, ((0, 0), (0, 1)))
    k1a = jnp.zeros((4, KER_HID), jnp.float32).at[:NDIM, :].set(K1[:NDIM])
    k1b = jnp.zeros((4, KER_HID), jnp.float32).at[:NDIM, :].set(K1[NDIM:])

    table = _proj_pack(x, gridp, W1, b1, W2, b2, k1a)
    tg = _gather(neighbors_index, table)

    ogp = jnp.pad(output_grid, ((0, 0), (0, 1)))
    out = _edge(tg, ogp, k1b, kb1, K2, kb2)
    return out[None]


# speedup vs baseline: 11.9992x; 1.3846x over previous
import functools

import jax
import jax.numpy as jnp
from jax import lax
from jax.experimental import pallas as pl
from jax.experimental.pallas import tpu as pltpu
from jax.experimental.pallas import tpu_sc as plsc

N_IN = 10000
N_OUT = 10000
K = 32
E = N_OUT * K
IN_DIM = 128
OUT_DIM = 128
PROJ_HID = 256
KER_HID = 32
NDIM = 3
TW = 128

_PROJ_BLK = 1000


def _proj_body(x_ref, g_ref, w1_ref, b1_ref, w2_ref, b2_ref, k1a_ref, o_ref):
    h = jnp.dot(x_ref[...], w1_ref[...], preferred_element_type=jnp.float32)
    h = jax.nn.gelu(h + b1_ref[...])
    f = jnp.dot(h, w2_ref[...], preferred_element_type=jnp.float32) + b2_ref[...]
    p = jnp.dot(g_ref[...], k1a_ref[...], preferred_element_type=jnp.float32)
    fb = f.astype(jnp.bfloat16).astype(jnp.float32)
    pb = p.astype(jnp.bfloat16).astype(jnp.float32)
    pe = jnp.concatenate(
        [pb, jnp.zeros((_PROJ_BLK, TW - KER_HID), jnp.float32)], axis=1)
    u = (lax.bitcast_convert_type(fb, jnp.uint32)
         | (lax.bitcast_convert_type(pe, jnp.uint32) >> 16))
    o_ref[...] = lax.bitcast_convert_type(u, jnp.float32)


def _proj_pack(x, gridp, w1, b1, w2, b2, k1a):
    grid = (N_IN // _PROJ_BLK,)
    return pl.pallas_call(
        _proj_body,
        grid=grid,
        in_specs=[
            pl.BlockSpec((_PROJ_BLK, IN_DIM), lambda i: (i, 0)),
            pl.BlockSpec((_PROJ_BLK, 4), lambda i: (i, 0)),
            pl.BlockSpec((IN_DIM, PROJ_HID), lambda i: (0, 0)),
            pl.BlockSpec((PROJ_HID,), lambda i: (0,)),
            pl.BlockSpec((PROJ_HID, OUT_DIM), lambda i: (0, 0)),
            pl.BlockSpec((OUT_DIM,), lambda i: (0,)),
            pl.BlockSpec((4, KER_HID), lambda i: (0, 0)),
        ],
        out_specs=pl.BlockSpec((_PROJ_BLK, TW), lambda i: (i, 0)),
        out_shape=jax.ShapeDtypeStruct((N_IN, TW), jnp.float32),
    )(x, gridp, w1, b1, w2, b2, k1a)


_NC = 2
_NS = 16
_NW = _NC * _NS
_NSPLIT = 5
_E_S = E // _NSPLIT
_PTS_S = N_OUT // _NSPLIT
_PER_W = _E_S // _NW
_CHUNK = 80
_NCHUNK = _PER_W // _CHUNK
_NBUF = 4


def _gather_body(idx_hbm, t_hbm, t_out, idx_v, t_v, sg, so):
    wid = lax.axis_index("s") * _NC + lax.axis_index("c")
    base = wid * _PER_W

    pltpu.sync_copy(idx_hbm.at[pl.ds(base, _PER_W)], idx_v)

    def start_gather(t, buf):
        pltpu.async_copy(
            t_hbm.at[idx_v.at[pl.ds(t * _CHUNK, _CHUNK)]], t_v.at[buf], sg)

    def drain_gather(buf):
        pltpu.make_async_copy(
            t_hbm.at[idx_v.at[pl.ds(0, _CHUNK)]], t_v.at[buf], sg).wait()

    def drain_store(buf):
        pltpu.make_async_copy(
            t_v.at[buf], t_out.at[pl.ds(base, _CHUNK)], so).wait()

    start_gather(0, 0)
    start_gather(1, 1)

    def body(t, carry):
        cur = lax.rem(t, _NBUF)

        @pl.when(t + 2 < _NCHUNK)
        def _():
            nxt = lax.rem(t + 2, _NBUF)

            @pl.when(t >= 2)
            def _():
                drain_store(nxt)
            start_gather(t + 2, nxt)

        drain_gather(cur)
        pltpu.async_copy(
            t_v.at[cur], t_out.at[pl.ds(base + t * _CHUNK, _CHUNK)], so)
        return carry

    lax.fori_loop(0, _NCHUNK, body, 0)
    for b in range(_NBUF):
        drain_store(b)


def _gather(idx, table):
    mesh = plsc.VectorSubcoreMesh(core_axis_name="c", subcore_axis_name="s")
    return pl.kernel(
        _gather_body,
        out_type=jax.ShapeDtypeStruct((_E_S, TW), jnp.float32),
        mesh=mesh,
        scratch_types=[
            pltpu.VMEM((_PER_W,), jnp.int32),
            pltpu.VMEM((_NBUF, _CHUNK, TW), jnp.float32),
            pltpu.SemaphoreType.DMA,
            pltpu.SemaphoreType.DMA,
        ],
    )(idx, table)


_BPTS = 400
_BE = _BPTS * K


def _edge_body(t_ref, og_ref, k1b_ref, kb1_ref, k2_ref, kb2_ref, o_ref):
    u = lax.bitcast_convert_type(t_ref[...], jnp.uint32)
    g = lax.bitcast_convert_type(u & jnp.uint32(0xFFFF0000), jnp.float32)
    a1 = lax.bitcast_convert_type(u << 16, jnp.float32)[:, :KER_HID]
    q = (
        jnp.dot(og_ref[...], k1b_ref[...], preferred_element_type=jnp.float32)
        + kb1_ref[...]
    )
    h1 = jax.nn.gelu(a1.reshape(_BPTS, K, KER_HID) + q[:, None, :])
    kern = (
        jnp.dot(h1.reshape(_BE, KER_HID), k2_ref[...],
                preferred_element_type=jnp.float32)
        + kb2_ref[...]
    )
    vals = kern * g
    o_ref[...] = vals.reshape(_BPTS, K, OUT_DIM).sum(axis=1) * (1.0 / K)


def _edge(tgb, ogp, k1b, kb1, k2, kb2):
    grid = (_PTS_S // _BPTS,)
    return pl.pallas_call(
        _edge_body,
        grid=grid,
        in_specs=[
            pl.BlockSpec((_BE, TW), lambda i: (i, 0)),
            pl.BlockSpec((_BPTS, 4), lambda i: (i, 0)),
            pl.BlockSpec((4, KER_HID), lambda i: (0, 0)),
            pl.BlockSpec((KER_HID,), lambda i: (0,)),
            pl.BlockSpec((KER_HID, OUT_DIM), lambda i: (0, 0)),
            pl.BlockSpec((OUT_DIM,), lambda i: (0,)),
        ],
        out_specs=pl.BlockSpec((_BPTS, OUT_DIM), lambda i: (i, 0)),
        out_shape=jax.ShapeDtypeStruct((_PTS_S, OUT_DIM), jnp.float32),
    )(tgb, ogp, k1b, kb1, k2, kb2)


def kernel(inp, input_grid, output_grid, neighbors_index,
           W1, b1, W2, b2, K1, kb1, K2, kb2):
    x = inp[0]
    gridp = jnp.pad(input_grid, ((0, 0), (0, 1)))
    k1a = jnp.zeros((4, KER_HID), jnp.float32).at[:NDIM, :].set(K1[:NDIM])
    k1b = jnp.zeros((4, KER_HID), jnp.float32).at[:NDIM, :].set(K1[NDIM:])

    table = _proj_pack(x, gridp, W1, b1, W2, b2, k1a)
    ogp = jnp.pad(output_grid, ((0, 0), (0, 1)))

    outs = []
    for i in range(_NSPLIT):
        idx_i = lax.slice_in_dim(neighbors_index, i * _E_S, (i + 1) * _E_S)
        og_i = lax.slice_in_dim(ogp, i * _PTS_S, (i + 1) * _PTS_S)
        tg = _gather(idx_i, table)
        outs.append(_edge(tg, og_i, k1b, kb1, K2, kb2))
    out = jnp.concatenate(outs, axis=0)
    return out[None]
